# baseline (speedup 1.0000x reference)
import functools

import jax
import jax.numpy as jnp
from jax import lax
from jax.experimental import pallas as pl
from jax.experimental.pallas import tpu as pltpu
from jax.experimental.pallas import tpu_sc as plsc

N_NODES = 10000
N_EDGES = 320000
D = 128

NC, NS = 2, 16
NW = NC * NS
NB = 320
NPAD = NW * NB
CAP = 16384
ECH = 8000
G = 96
PK = 512

_MESH = plsc.VectorSubcoreMesh(
    core_axis_name="c", subcore_axis_name="s", num_cores=NC, num_subcores=NS)


def _worker_id():
    return lax.axis_index("s") * NC + lax.axis_index("c")


def _filter_body(edge_hbm, pkl_hbm, cnt_hbm,
                 s0, s1, d0, d1, pkout, cbuf, semA, semB):
    w = _worker_id()
    lo = w * NB
    nch = N_EDGES // ECH

    def stage(c, sb, db, sem):
        pltpu.async_copy(edge_hbm.at[pl.ds(c * ECH, ECH)], sb, sem)
        pltpu.async_copy(edge_hbm.at[pl.ds(N_EDGES + c * ECH, ECH)], db, sem)

    def drain(c, sb, db, sem):
        pltpu.make_async_copy(edge_hbm.at[pl.ds(c * ECH, ECH)],
                              sb, sem).wait()
        pltpu.make_async_copy(edge_hbm.at[pl.ds(N_EDGES + c * ECH, ECH)],
                              db, sem).wait()

    def process(sb, db, cnt):
        def vec(i, cnt):
            d = db[pl.ds(i * 16, 16)]
            s = sb[pl.ds(i * 16, 16)]
            m = (d >= lo) & (d < lo + NB) & (cnt < CAP - 512)
            mi = m.astype(jnp.int32)
            inc = plsc.cumsum(mi)
            pos = cnt + inc - mi
            plsc.store_scatter(pkout, [pos], s * PK + (d - lo), mask=m)
            return cnt + inc[15]
        return lax.fori_loop(0, ECH // 16, vec, cnt)

    stage(0, s0, d0, semA)

    def pair(p, cnt):
        c0 = 2 * p
        stage(c0 + 1, s1, d1, semB)
        drain(c0, s0, d0, semA)
        cnt = process(s0, d0, cnt)

        @pl.when(c0 + 2 < nch)
        def _():
            stage(c0 + 2, s0, d0, semA)

        drain(c0 + 1, s1, d1, semB)
        return process(s1, d1, cnt)

    cnt = lax.fori_loop(0, nch // 2, pair, jnp.int32(0))

    dump = jnp.full((16,), NB, jnp.int32)
    for j in range(0, G + 16, 16):
        pkout[pl.ds(cnt + j, 16)] = dump

    cbuf[...] = jnp.broadcast_to(cnt, (16,))
    pltpu.sync_copy(cbuf, cnt_hbm.at[w])
    pltpu.sync_copy(pkout, pkl_hbm.at[w])


_filter_call = pl.kernel(
    _filter_body,
    out_type=(
        jax.ShapeDtypeStruct((NW, CAP), jnp.int32),
        jax.ShapeDtypeStruct((NW, 16), jnp.int32),
    ),
    mesh=_MESH,
    compiler_params=pltpu.CompilerParams(needs_layout_passes=False),
    scratch_types=[
        pltpu.VMEM((ECH,), jnp.int32),
        pltpu.VMEM((ECH,), jnp.int32),
        pltpu.VMEM((ECH,), jnp.int32),
        pltpu.VMEM((ECH,), jnp.int32),
        pltpu.VMEM((CAP,), jnp.int32),
        pltpu.VMEM((16,), jnp.int32),
        pltpu.SemaphoreType.DMA,
        pltpu.SemaphoreType.DMA,
    ],
)


def _segmax_body(b_hbm, pkl_hbm, cnt_hbm, out_hbm,
                 acc, acc2, rows0, rows1, pkfull, idx0, idx1, dst0, dst1,
                 cbuf, sem0, sem1):
    w = _worker_id()

    neg = jnp.full((16,), -jnp.inf, jnp.float32)

    def init(i, _):
        acc[pl.ds(i * 16, 16)] = neg
        acc2[pl.ds(i * 16, 16)] = neg
        return 0
    lax.fori_loop(0, (NB + 1) * D // 16, init, 0)

    pltpu.sync_copy(cnt_hbm.at[w], cbuf)
    pltpu.sync_copy(pkl_hbm.at[w], pkfull)
    k = cbuf[...][0]
    nchunks = (k + (G - 1)) // G

    def stage(g, idxb, dstb, rowsb, sem):
        for j in range(G // 16):
            p = pkfull[pl.ds(g * G + j * 16, 16)]
            idxb[pl.ds(j * 16, 16)] = p // PK
            dstb[pl.ds(j * 16, 16)] = lax.rem(p, PK) * D
        pltpu.async_copy(b_hbm.at[idxb], rowsb, sem)

    def process(idxb, dstb, rowsb, sem):
        pltpu.make_async_copy(b_hbm.at[idxb], rowsb, sem).wait()

        def block(j, _):
            dvec = dstb[pl.ds(j * 16, 16)]
            for l in range(0, 16, 4):
                lds = [dvec[l + t] for t in range(4)]
                eis = [j * 16 + l + t for t in range(4)]
                for f in range(D // 16):
                    vs = [rowsb[eis[t], pl.ds(f * 16, 16)] for t in range(4)]
                    a0 = acc[pl.ds(lds[0] + f * 16, 16)]
                    a1 = acc2[pl.ds(lds[1] + f * 16, 16)]
                    acc[pl.ds(lds[0] + f * 16, 16)] = jnp.maximum(a0, vs[0])
                    acc2[pl.ds(lds[1] + f * 16, 16)] = jnp.maximum(a1, vs[1])
                    a2 = acc[pl.ds(lds[2] + f * 16, 16)]
                    a3 = acc2[pl.ds(lds[3] + f * 16, 16)]
                    acc[pl.ds(lds[2] + f * 16, 16)] = jnp.maximum(a2, vs[2])
                    acc2[pl.ds(lds[3] + f * 16, 16)] = jnp.maximum(a3, vs[3])
            return 0
        lax.fori_loop(0, G // 16, block, 0)

    @pl.when(nchunks > 0)
    def _():
        stage(0, idx0, dst0, rows0, sem0)

    def pair(p, _):
        c0 = 2 * p

        @pl.when(c0 + 1 < nchunks)
        def _():
            stage(c0 + 1, idx1, dst1, rows1, sem1)

        process(idx0, dst0, rows0, sem0)

        @pl.when(c0 + 1 < nchunks)
        def _():
            @pl.when(c0 + 2 < nchunks)
            def _():
                stage(c0 + 2, idx0, dst0, rows0, sem0)

            process(idx1, dst1, rows1, sem1)
        return 0

    lax.fori_loop(0, (nchunks + 1) // 2, pair, 0)

    def merge(i, _):
        a = acc[pl.ds(i * 16, 16)]
        b = acc2[pl.ds(i * 16, 16)]
        acc[pl.ds(i * 16, 16)] = jnp.maximum(a, b)
        return 0
    lax.fori_loop(0, NB * D // 16, merge, 0)
    pltpu.sync_copy(acc.at[pl.ds(0, NB * D)],
                    out_hbm.at[pl.ds(w * NB * D, NB * D)])


_segmax_call = pl.kernel(
    _segmax_body,
    out_type=jax.ShapeDtypeStruct((NPAD * D,), jnp.float32),
    mesh=_MESH,
    compiler_params=pltpu.CompilerParams(needs_layout_passes=False),
    scratch_types=[
        pltpu.VMEM(((NB + 1) * D,), jnp.float32),
        pltpu.VMEM(((NB + 1) * D,), jnp.float32),
        pltpu.VMEM((G, D), jnp.float32),
        pltpu.VMEM((G, D), jnp.float32),
        pltpu.VMEM((CAP,), jnp.int32),
        pltpu.VMEM((G,), jnp.int32),
        pltpu.VMEM((G,), jnp.int32),
        pltpu.VMEM((G,), jnp.int32),
        pltpu.VMEM((G,), jnp.int32),
        pltpu.VMEM((16,), jnp.int32),
        pltpu.SemaphoreType.DMA,
        pltpu.SemaphoreType.DMA,
    ],
)


_BLK = 1000


def _dense1_body(x_ref, w1_ref, b1_ref, a_ref, b_ref):
    x = x_ref[...]
    wt = w1_ref[:D, :]
    wb = w1_ref[D:, :]
    a_ref[...] = jnp.dot(x, wt - wb, preferred_element_type=jnp.float32) \
        + b1_ref[...]
    b_ref[...] = jnp.dot(x, wb, preferred_element_type=jnp.float32)


def _dense2_body(a1_ref, seg_ref, w2_ref, b2_ref, a_ref, b_ref):
    t = a1_ref[...] + seg_ref[...]
    h = jnp.where(jnp.isfinite(t), jnp.maximum(t, 0.0), 0.0)
    wt = w2_ref[:D, :]
    wb = w2_ref[D:, :]
    a_ref[...] = jnp.dot(h, wt - wb, preferred_element_type=jnp.float32) \
        + b2_ref[...]
    b_ref[...] = jnp.dot(h, wb, preferred_element_type=jnp.float32)


def _final_body(a2_ref, seg_ref, o_ref):
    t = a2_ref[...] + seg_ref[...]
    o = jnp.where(jnp.isfinite(t), t, 0.0)
    m = jnp.max(o, axis=1, keepdims=True)
    ls = jnp.log(jnp.sum(jnp.exp(o - m), axis=1, keepdims=True))
    o_ref[...] = o - m - ls


def _row_spec():
    return pl.BlockSpec((_BLK, D), lambda i: (i, 0))


def _full_spec(shape):
    return pl.BlockSpec(shape, lambda i: tuple(0 for _ in shape))


_dense1 = pl.pallas_call(
    _dense1_body,
    grid=(N_NODES // _BLK,),
    in_specs=[_row_spec(), _full_spec((2 * D, D)), _full_spec((1, D))],
    out_specs=[_row_spec(), _row_spec()],
    out_shape=[jax.ShapeDtypeStruct((N_NODES, D), jnp.float32)] * 2,
)

_dense2 = pl.pallas_call(
    _dense2_body,
    grid=(N_NODES // _BLK,),
    in_specs=[_row_spec(), _row_spec(), _full_spec((2 * D, D)),
              _full_spec((1, D))],
    out_specs=[_row_spec(), _row_spec()],
    out_shape=[jax.ShapeDtypeStruct((N_NODES, D), jnp.float32)] * 2,
)

_final = pl.pallas_call(
    _final_body,
    grid=(N_NODES // _BLK,),
    in_specs=[_row_spec(), _row_spec()],
    out_specs=_row_spec(),
    out_shape=jax.ShapeDtypeStruct((N_NODES, D), jnp.float32),
)


def kernel(x, edge_index, W1, b1, W2, b2):
    pkl, cnts = _filter_call(edge_index.reshape(2 * N_EDGES))
    a1, bb1 = _dense1(x, W1, b1.reshape(1, D))
    seg1 = _segmax_call(bb1, pkl, cnts).reshape(NPAD, D)
    a2, bb2 = _dense2(a1, seg1[:N_NODES], W2, b2.reshape(1, D))
    seg2 = _segmax_call(bb2, pkl, cnts).reshape(NPAD, D)
    return _final(a2, seg2[:N_NODES])


# speedup vs baseline: 1.1905x; 1.1905x over previous
import functools

import jax
import jax.numpy as jnp
from jax import lax
from jax.experimental import pallas as pl
from jax.experimental.pallas import tpu as pltpu
from jax.experimental.pallas import tpu_sc as plsc

N_NODES = 10000
N_EDGES = 320000
D = 128

NC, NS = 2, 16
NW = NC * NS
NB = 320
NPAD = NW * NB
CAP = 16384
ECH = 8000
G = 96
PK = 512

_MESH = plsc.VectorSubcoreMesh(
    core_axis_name="c", subcore_axis_name="s", num_cores=NC, num_subcores=NS)


def _worker_id():
    return lax.axis_index("s") * NC + lax.axis_index("c")


def _filter_body(edge_hbm, pkl_hbm, cnt_hbm,
                 s0, s1, d0, d1, pkout, pkcmp, cbuf, semA, semB):
    w = _worker_id()
    lo = w * NB
    nch = N_EDGES // ECH
    iota = lax.iota(jnp.int32, 16)
    iota16 = iota * 16
    CAPL16 = (CAP // 16 - 8) * 16

    def stage(c, sb, db, sem):
        pltpu.async_copy(edge_hbm.at[pl.ds(c * ECH, ECH)], sb, sem)
        pltpu.async_copy(edge_hbm.at[pl.ds(N_EDGES + c * ECH, ECH)], db, sem)

    def drain(c, sb, db, sem):
        pltpu.make_async_copy(edge_hbm.at[pl.ds(c * ECH, ECH)],
                              sb, sem).wait()
        pltpu.make_async_copy(edge_hbm.at[pl.ds(N_EDGES + c * ECH, ECH)],
                              db, sem).wait()

    def process(sb, db, cvec16):
        def vec(i, cvec16):
            d = db[pl.ds(i * 16, 16)]
            s = sb[pl.ds(i * 16, 16)]
            m = (d >= lo) & (d < lo + NB) & (cvec16 < CAPL16)
            mi = m.astype(jnp.int32)
            plsc.store_scatter(pkout, [cvec16 + iota],
                               s * PK + (d - lo), mask=m)
            return cvec16 + mi * 16
        return lax.fori_loop(0, ECH // 16, vec, cvec16)

    stage(0, s0, d0, semA)

    def pair(p, cvec16):
        c0 = 2 * p
        stage(c0 + 1, s1, d1, semB)
        drain(c0, s0, d0, semA)
        cvec16 = process(s0, d0, cvec16)

        @pl.when(c0 + 2 < nch)
        def _():
            stage(c0 + 2, s0, d0, semA)

        drain(c0 + 1, s1, d1, semB)
        return process(s1, d1, cvec16)

    cvec16 = lax.fori_loop(0, nch // 2, pair, jnp.zeros((16,), jnp.int32))

    cnt = jnp.int32(0)
    for l in range(16):
        cl = cvec16[l] // 16
        nq = (cl + 15) // 16

        def qloop(q, carry, l=l):
            src_idx = iota16 + (q * 256 + l)
            vals = plsc.load_gather(pkout, [src_idx])
            pkcmp[pl.ds(carry + q * 16, 16)] = vals
            return carry

        lax.fori_loop(0, nq, qloop, cnt)
        cnt = cnt + cl

    dump = jnp.full((16,), NB, jnp.int32)
    for j in range(0, G + 16, 16):
        pkcmp[pl.ds(cnt + j, 16)] = dump

    cbuf[...] = jnp.broadcast_to(cnt, (16,))
    pltpu.sync_copy(cbuf, cnt_hbm.at[w])
    pltpu.sync_copy(pkcmp, pkl_hbm.at[w])


_filter_call = pl.kernel(
    _filter_body,
    out_type=(
        jax.ShapeDtypeStruct((NW, CAP), jnp.int32),
        jax.ShapeDtypeStruct((NW, 16), jnp.int32),
    ),
    mesh=_MESH,
    compiler_params=pltpu.CompilerParams(needs_layout_passes=False),
    scratch_types=[
        pltpu.VMEM((ECH,), jnp.int32),
        pltpu.VMEM((ECH,), jnp.int32),
        pltpu.VMEM((ECH,), jnp.int32),
        pltpu.VMEM((ECH,), jnp.int32),
        pltpu.VMEM((CAP,), jnp.int32),
        pltpu.VMEM((CAP,), jnp.int32),
        pltpu.VMEM((16,), jnp.int32),
        pltpu.SemaphoreType.DMA,
        pltpu.SemaphoreType.DMA,
    ],
)


def _segmax_body(b_hbm, pkl_hbm, cnt_hbm, out_hbm,
                 acc, acc2, rows0, rows1, pkfull, idx0, idx1, dst0, dst1,
                 cbuf, sem0, sem1):
    w = _worker_id()

    neg = jnp.full((16,), -jnp.inf, jnp.float32)

    def init(i, _):
        acc[pl.ds(i * 16, 16)] = neg
        acc2[pl.ds(i * 16, 16)] = neg
        return 0
    lax.fori_loop(0, (NB + 1) * D // 16, init, 0)

    pltpu.sync_copy(cnt_hbm.at[w], cbuf)
    pltpu.sync_copy(pkl_hbm.at[w], pkfull)
    k = cbuf[...][0]
    nchunks = (k + (G - 1)) // G

    def stage(g, idxb, dstb, rowsb, sem):
        for j in range(G // 16):
            p = pkfull[pl.ds(g * G + j * 16, 16)]
            idxb[pl.ds(j * 16, 16)] = p // PK
            dstb[pl.ds(j * 16, 16)] = lax.rem(p, PK) * D
        pltpu.async_copy(b_hbm.at[idxb], rowsb, sem)

    def process(idxb, dstb, rowsb, sem):
        pltpu.make_async_copy(b_hbm.at[idxb], rowsb, sem).wait()

        def block(j, _):
            dvec = dstb[pl.ds(j * 16, 16)]
            for l in range(0, 16, 4):
                lds = [dvec[l + t] for t in range(4)]
                eis = [j * 16 + l + t for t in range(4)]
                for f in range(D // 16):
                    vs = [rowsb[eis[t], pl.ds(f * 16, 16)] for t in range(4)]
                    a0 = acc[pl.ds(lds[0] + f * 16, 16)]
                    a1 = acc2[pl.ds(lds[1] + f * 16, 16)]
                    acc[pl.ds(lds[0] + f * 16, 16)] = jnp.maximum(a0, vs[0])
                    acc2[pl.ds(lds[1] + f * 16, 16)] = jnp.maximum(a1, vs[1])
                    a2 = acc[pl.ds(lds[2] + f * 16, 16)]
                    a3 = acc2[pl.ds(lds[3] + f * 16, 16)]
                    acc[pl.ds(lds[2] + f * 16, 16)] = jnp.maximum(a2, vs[2])
                    acc2[pl.ds(lds[3] + f * 16, 16)] = jnp.maximum(a3, vs[3])
            return 0
        lax.fori_loop(0, G // 16, block, 0)

    @pl.when(nchunks > 0)
    def _():
        stage(0, idx0, dst0, rows0, sem0)

    def pair(p, _):
        c0 = 2 * p

        @pl.when(c0 + 1 < nchunks)
        def _():
            stage(c0 + 1, idx1, dst1, rows1, sem1)

        process(idx0, dst0, rows0, sem0)

        @pl.when(c0 + 1 < nchunks)
        def _():
            @pl.when(c0 + 2 < nchunks)
            def _():
                stage(c0 + 2, idx0, dst0, rows0, sem0)

            process(idx1, dst1, rows1, sem1)
        return 0

    lax.fori_loop(0, (nchunks + 1) // 2, pair, 0)

    def merge(i, _):
        a = acc[pl.ds(i * 16, 16)]
        b = acc2[pl.ds(i * 16, 16)]
        acc[pl.ds(i * 16, 16)] = jnp.maximum(a, b)
        return 0
    lax.fori_loop(0, NB * D // 16, merge, 0)
    pltpu.sync_copy(acc.at[pl.ds(0, NB * D)],
                    out_hbm.at[pl.ds(w * NB * D, NB * D)])


_segmax_call = pl.kernel(
    _segmax_body,
    out_type=jax.ShapeDtypeStruct((NPAD * D,), jnp.float32),
    mesh=_MESH,
    compiler_params=pltpu.CompilerParams(needs_layout_passes=False),
    scratch_types=[
        pltpu.VMEM(((NB + 1) * D,), jnp.float32),
        pltpu.VMEM(((NB + 1) * D,), jnp.float32),
        pltpu.VMEM((G, D), jnp.float32),
        pltpu.VMEM((G, D), jnp.float32),
        pltpu.VMEM((CAP,), jnp.int32),
        pltpu.VMEM((G,), jnp.int32),
        pltpu.VMEM((G,), jnp.int32),
        pltpu.VMEM((G,), jnp.int32),
        pltpu.VMEM((G,), jnp.int32),
        pltpu.VMEM((16,), jnp.int32),
        pltpu.SemaphoreType.DMA,
        pltpu.SemaphoreType.DMA,
    ],
)


_BLK = 1000


def _dense1_body(x_ref, w1_ref, b1_ref, a_ref, b_ref):
    x = x_ref[...]
    wt = w1_ref[:D, :]
    wb = w1_ref[D:, :]
    a_ref[...] = jnp.dot(x, wt - wb, preferred_element_type=jnp.float32) \
        + b1_ref[...]
    b_ref[...] = jnp.dot(x, wb, preferred_element_type=jnp.float32)


def _dense2_body(a1_ref, seg_ref, w2_ref, b2_ref, a_ref, b_ref):
    t = a1_ref[...] + seg_ref[...]
    h = jnp.where(jnp.isfinite(t), jnp.maximum(t, 0.0), 0.0)
    wt = w2_ref[:D, :]
    wb = w2_ref[D:, :]
    a_ref[...] = jnp.dot(h, wt - wb, preferred_element_type=jnp.float32) \
        + b2_ref[...]
    b_ref[...] = jnp.dot(h, wb, preferred_element_type=jnp.float32)


def _final_body(a2_ref, seg_ref, o_ref):
    t = a2_ref[...] + seg_ref[...]
    o = jnp.where(jnp.isfinite(t), t, 0.0)
    m = jnp.max(o, axis=1, keepdims=True)
    ls = jnp.log(jnp.sum(jnp.exp(o - m), axis=1, keepdims=True))
    o_ref[...] = o - m - ls


def _row_spec():
    return pl.BlockSpec((_BLK, D), lambda i: (i, 0))


def _full_spec(shape):
    return pl.BlockSpec(shape, lambda i: tuple(0 for _ in shape))


_dense1 = pl.pallas_call(
    _dense1_body,
    grid=(N_NODES // _BLK,),
    in_specs=[_row_spec(), _full_spec((2 * D, D)), _full_spec((1, D))],
    out_specs=[_row_spec(), _row_spec()],
    out_shape=[jax.ShapeDtypeStruct((N_NODES, D), jnp.float32)] * 2,
)

_dense2 = pl.pallas_call(
    _dense2_body,
    grid=(N_NODES // _BLK,),
    in_specs=[_row_spec(), _row_spec(), _full_spec((2 * D, D)),
              _full_spec((1, D))],
    out_specs=[_row_spec(), _row_spec()],
    out_shape=[jax.ShapeDtypeStruct((N_NODES, D), jnp.float32)] * 2,
)

_final = pl.pallas_call(
    _final_body,
    grid=(N_NODES // _BLK,),
    in_specs=[_row_spec(), _row_spec()],
    out_specs=_row_spec(),
    out_shape=jax.ShapeDtypeStruct((N_NODES, D), jnp.float32),
)


def kernel(x, edge_index, W1, b1, W2, b2):
    pkl, cnts = _filter_call(edge_index.reshape(2 * N_EDGES))
    a1, bb1 = _dense1(x, W1, b1.reshape(1, D))
    seg1 = _segmax_call(bb1, pkl, cnts).reshape(NPAD, D)
    a2, bb2 = _dense2(a1, seg1[:N_NODES], W2, b2.reshape(1, D))
    seg2 = _segmax_call(bb2, pkl, cnts).reshape(NPAD, D)
    return _final(a2, seg2[:N_NODES])
